# baseline (speedup 1.0000x reference)
import functools

import jax
import jax.numpy as jnp
from jax import lax
from jax.experimental import pallas as pl
from jax.experimental.pallas import tpu as pltpu
from jax.experimental.pallas import tpu_sc as plsc

_BSZ = 64
_BEAM = 8
_VOCAB = 100000
_K = 16
_NC = 2
_NS = 16
_NW = _NC * _NS
_ROWS_PER_W = _BSZ // _NW
_MAIN = 99968
_TAIL = _VOCAB - _MAIN
_VCH = 1408
_NCH = _MAIN // _VCH
_NGRP = _BEAM * (_VCH // 128)


def _splat(x):
    return lax.broadcast(x, (16,))


def _gather16(vec, idx):
    return lax.gather(
        vec, idx[:, None],
        lax.GatherDimensionNumbers(
            offset_dims=(), collapsed_slice_dims=(0,), start_index_map=(0,)),
        slice_sizes=(1,),
        mode=lax.GatherScatterMode.PROMISE_IN_BOUNDS)


def _topk_body(lp, tail, bias, out_s, out_i, out_b,
               buf0, buf1, buf2, buf3, tbuf, bias16, cur_v, cur_i, th_b,
               st_s, st_i, st_b, sem0, sem1, sem2, sem3):
    w = lax.axis_index("s") * _NC + lax.axis_index("c")

    pltpu.sync_copy(bias.at[pl.ds(w * _ROWS_PER_W * _BEAM, 16)], bias16)

    def merge(vb, idx_base):
        iv = _splat(idx_base) + lax.iota(jnp.int32, 16)
        sk, si = plsc.sort_key_val(vb, iv)
        cv = cur_v[...]
        ci = cur_i[...]
        rk = lax.rev(sk, (0,))
        ri = lax.rev(si, (0,))
        take = rk > cv
        mk = jnp.where(take, rk, cv)
        mi = jnp.where(take, ri, ci)
        nk, ni = plsc.sort_key_val(mk, mi)
        cur_v[...] = nk
        cur_i[...] = ni
        th_b[...] = _splat(jnp.min(nk))

    def consider(vj, bias_sp, idx_base):
        th = th_b[...] - bias_sp

        @pl.when(jnp.any(vj > th))
        def _():
            merge(vj + bias_sp, idx_base)

    def process(buf, sb, c):
        v0 = c * _VCH
        ntile = _VCH // 128

        def ebody(e, carry):
            bias_sp = _gather16(bias16[...], _splat(sb * _BEAM + e))
            m = buf[e, pl.ds(0, 16)]
            for q in range(1, ntile * 8):
                m = jnp.maximum(m, buf[e, pl.ds(16 * q, 16)])
            th = th_b[...] - bias_sp

            @pl.when(jnp.any(m > th))
            def _():
                def gbody(g, carry2):
                    base = g * 128
                    vs = [buf[e, pl.ds(base + 16 * j, 16)] for j in range(8)]
                    mg = vs[0]
                    for j in range(1, 8):
                        mg = jnp.maximum(mg, vs[j])
                    thg = th_b[...] - bias_sp

                    @pl.when(jnp.any(mg > thg))
                    def _():
                        fbase = e * _VOCAB + v0 + base
                        for j in range(8):
                            consider(vs[j], bias_sp, fbase + 16 * j)

                    return carry2

                lax.fori_loop(0, ntile, gbody, 0)

            return carry

        lax.fori_loop(0, _BEAM, ebody, 0)

    for sb in range(_ROWS_PER_W):
        row = w * _ROWS_PER_W + sb
        neg = jnp.full((16,), -jnp.inf, jnp.float32)
        cur_v[...] = neg
        cur_i[...] = jnp.zeros((16,), jnp.int32)
        th_b[...] = neg

        def chunk_src(c, row=row):
            return lp.at[row, :, pl.ds(c * _VCH, _VCH)]

        bufs = (buf0, buf1, buf2, buf3)
        sems = (sem0, sem1, sem2, sem3)
        for k in range(4):
            pltpu.make_async_copy(chunk_src(jnp.int32(k)), bufs[k],
                                  sems[k]).start()

        def hbody(h, carry, row=row):
            def chunk_src(c):
                return lp.at[row, :, pl.ds(c * _VCH, _VCH)]

            for k in range(4):
                c = 4 * h + k
                cc = jnp.minimum(c, _NCH - 1)
                pltpu.make_async_copy(chunk_src(cc), bufs[k], sems[k]).wait()

                @pl.when(c <= _NCH - 1)
                def _(k=k, c=c):
                    process(bufs[k], carry, c)

                nxt = jnp.minimum(c + 4, _NCH - 1)
                pltpu.make_async_copy(chunk_src(nxt), bufs[k], sems[k]).start()

            return carry

        lax.fori_loop(0, (_NCH + 3) // 4, hbody, sb)
        for k in range(4):
            pltpu.make_async_copy(chunk_src(jnp.int32(0)), bufs[k],
                                  sems[k]).wait()

        pltpu.sync_copy(tail.at[row], tbuf)
        for e in range(_BEAM):
            bias_sp = _gather16(bias16[...], _splat(sb * _BEAM + e))
            for jj in range(_TAIL // 16):
                vj = tbuf[e, pl.ds(16 * jj, 16)]
                consider(vj, bias_sp, e * _VOCAB + _MAIN + 16 * jj)

        dv = lax.rev(cur_v[...], (0,))
        di = lax.rev(cur_i[...], (0,))
        bm = jnp.zeros((16,), jnp.int32)
        for t in range(1, _BEAM):
            bm = bm + jnp.where(di >= t * _VOCAB, 1, 0).astype(jnp.int32)
        ix = di - bm * _VOCAB
        st_s[...] = dv
        st_i[...] = ix
        st_b[...] = bm
        pltpu.sync_copy(st_s, out_s.at[row])
        pltpu.sync_copy(st_i, out_i.at[row])
        pltpu.sync_copy(st_b, out_b.at[row])


@functools.partial(
    pl.kernel,
    out_type=(
        jax.ShapeDtypeStruct((_BSZ, _K), jnp.float32),
        jax.ShapeDtypeStruct((_BSZ, _K), jnp.int32),
        jax.ShapeDtypeStruct((_BSZ, _K), jnp.int32),
    ),
    mesh=plsc.VectorSubcoreMesh(core_axis_name="c", subcore_axis_name="s"),
    compiler_params=pltpu.CompilerParams(needs_layout_passes=False),
    scratch_types=(
        pltpu.VMEM((_BEAM, _VCH), jnp.float32),
        pltpu.VMEM((_BEAM, _VCH), jnp.float32),
        pltpu.VMEM((_BEAM, _VCH), jnp.float32),
        pltpu.VMEM((_BEAM, _VCH), jnp.float32),
        pltpu.VMEM((_BEAM, _TAIL), jnp.float32),
        pltpu.VMEM((16,), jnp.float32),
        pltpu.VMEM((16,), jnp.float32),
        pltpu.VMEM((16,), jnp.int32),
        pltpu.VMEM((16,), jnp.float32),
        pltpu.VMEM((16,), jnp.float32),
        pltpu.VMEM((16,), jnp.int32),
        pltpu.VMEM((16,), jnp.int32),
        pltpu.SemaphoreType.DMA,
        pltpu.SemaphoreType.DMA,
        pltpu.SemaphoreType.DMA,
        pltpu.SemaphoreType.DMA,
    ),
)
def _topk_sc(lp, tail, bias, out_s, out_i, out_b, *scratch):
    _topk_body(lp, tail, bias, out_s, out_i, out_b, *scratch)


def kernel(lprobs, scores, step):
    bias = lax.dynamic_index_in_dim(scores, step - 1, axis=2, keepdims=False)
    tail = lax.slice(lprobs, (0, 0, _MAIN), (_BSZ, _BEAM, _VOCAB))
    return _topk_sc(lprobs, tail, bias.reshape(-1))
